# Initial kernel scaffold; baseline (speedup 1.0000x reference)
import jax
import jax.numpy as jnp
from jax import lax
from jax.experimental import pallas as pl
from jax.experimental.pallas import tpu as pltpu
from jax.experimental.pallas import tpu_sc as plsc

_N = 4096
_NW = 32
_QPW = _N // _NW
_L = 16
_NCHUNK = _N // _L
_UNROLL = 8


def _sqrt16(x):
    i = plsc.bitcast(x, jnp.int32)
    y = plsc.bitcast(jnp.int32(0x5F3759DF) - (i >> 1), jnp.float32)
    for _ in range(4):
        y = y * (1.5 - 0.5 * x * y * y)
    return x * y


def _body(xs_h, ys_h, zs_h, sx_h, sy_h, sz_h, ox_h, oy_h, oz_h,
          xs_v, ys_v, zs_v, g2_v, g3_v, g4_v,
          sx_v, sy_v, sz_v, ox_v, oy_v, oz_v):
    wid = lax.axis_index("s") * 2 + lax.axis_index("c")
    base = wid * _QPW

    pltpu.sync_copy(xs_h, xs_v)
    pltpu.sync_copy(ys_h, ys_v)
    pltpu.sync_copy(zs_h, zs_v)
    pltpu.sync_copy(sx_h.at[pl.ds(base, _QPW)], sx_v)
    pltpu.sync_copy(sy_h.at[pl.ds(base, _QPW)], sy_v)
    pltpu.sync_copy(sz_h.at[pl.ds(base, _QPW)], sz_v)

    iota = lax.iota(jnp.int32, _L)
    lane0 = iota == 0
    inf_v = jnp.full((_L,), jnp.inf, jnp.float32)

    def per_query(q, _):
        qidx = jnp.full((_L,), base + q, jnp.int32)
        qx = plsc.load_gather(xs_v, [qidx])
        qy = plsc.load_gather(ys_v, [qidx])
        qz = plsc.load_gather(zs_v, [qidx])

        def chunk_body(c, carry):
            m1, m2, m3, m4 = carry
            coff = c * (_UNROLL * _L)
            for u in range(_UNROLL):
                off = coff + u * _L
                x = xs_v[pl.ds(off, _L)]
                y = ys_v[pl.ds(off, _L)]
                z = zs_v[pl.ds(off, _L)]
                dx = qx - x
                dy = qy - y
                dz = qz - z
                t = dx * dx + dy * dy + dz * dz
                m1, t = jnp.minimum(m1, t), jnp.maximum(m1, t)
                m2, t = jnp.minimum(m2, t), jnp.maximum(m2, t)
                m3, t = jnp.minimum(m3, t), jnp.maximum(m3, t)
                m4 = jnp.minimum(m4, t)
            return m1, m2, m3, m4

        m1, m2, m3, m4 = lax.fori_loop(
            0, _NCHUNK // _UNROLL, chunk_body, (inf_v, inf_v, inf_v, inf_v))

        g = []
        for r in range(4):
            gr = jnp.min(m1)
            g.append(gr)
            if r < 3:
                ff = plsc.all_reduce_ffs(m1 == gr)
                sel = iota == ff
                m1 = jnp.where(sel, m2, m1)
                m2 = jnp.where(sel, m3, m2)
                m3 = jnp.where(sel, m4, m3)

        qfull = jnp.full((_L,), q, jnp.int32)
        plsc.store_scatter(g2_v, [qfull], jnp.full((_L,), g[1]), mask=lane0)
        plsc.store_scatter(g3_v, [qfull], jnp.full((_L,), g[2]), mask=lane0)
        plsc.store_scatter(g4_v, [qfull], jnp.full((_L,), g[3]), mask=lane0)
        return 0

    lax.fori_loop(0, _QPW, per_query, 0)

    for c in range(_QPW // _L):
        off = c * _L
        s = (_sqrt16(g2_v[pl.ds(off, _L)])
             + _sqrt16(g3_v[pl.ds(off, _L)])
             + _sqrt16(g4_v[pl.ds(off, _L)])) / 3.0
        s = jnp.maximum(s, 1e-5)
        ox_v[pl.ds(off, _L)] = s * sx_v[pl.ds(off, _L)]
        oy_v[pl.ds(off, _L)] = s * sy_v[pl.ds(off, _L)]
        oz_v[pl.ds(off, _L)] = s * sz_v[pl.ds(off, _L)]

    pltpu.sync_copy(ox_v, ox_h.at[pl.ds(base, _QPW)])
    pltpu.sync_copy(oy_v, oy_h.at[pl.ds(base, _QPW)])
    pltpu.sync_copy(oz_v, oz_h.at[pl.ds(base, _QPW)])


@jax.jit
def kernel(points, colors, scales):
    del colors
    xs = jnp.ascontiguousarray(points[:, 0])
    ys = jnp.ascontiguousarray(points[:, 1])
    zs = jnp.ascontiguousarray(points[:, 2])
    sx = jnp.ascontiguousarray(scales[:, 0])
    sy = jnp.ascontiguousarray(scales[:, 1])
    sz = jnp.ascontiguousarray(scales[:, 2])

    f32 = jnp.float32
    vec = jax.ShapeDtypeStruct((_N,), f32)
    run = pl.kernel(
        _body,
        out_type=(vec, vec, vec),
        mesh=plsc.VectorSubcoreMesh(core_axis_name="c", subcore_axis_name="s"),
        scratch_types=(
            pltpu.VMEM((_N,), f32),
            pltpu.VMEM((_N,), f32),
            pltpu.VMEM((_N,), f32),
            pltpu.VMEM((_QPW,), f32),
            pltpu.VMEM((_QPW,), f32),
            pltpu.VMEM((_QPW,), f32),
            pltpu.VMEM((_QPW,), f32),
            pltpu.VMEM((_QPW,), f32),
            pltpu.VMEM((_QPW,), f32),
            pltpu.VMEM((_QPW,), f32),
            pltpu.VMEM((_QPW,), f32),
            pltpu.VMEM((_QPW,), f32),
        ),
    )
    ox, oy, oz = run(xs, ys, zs, sx, sy, sz)
    return jnp.stack([ox, oy, oz], axis=1)


# speedup vs baseline: 40.6771x; 40.6771x over previous
import jax
import jax.numpy as jnp
from jax import lax
from jax.experimental import pallas as pl
from jax.experimental.pallas import tpu as pltpu
from jax.experimental.pallas import tpu_sc as plsc

_N = 4096
_NW = 32
_QPW = _N // _NW
_L = 16
_NCHUNK = _N // _L
_UNROLL = 8


def _sqrt16(x):
    i = plsc.bitcast(x, jnp.int32)
    y = plsc.bitcast(jnp.int32(0x5F3759DF) - (i >> 1), jnp.float32)
    for _ in range(4):
        y = y * (1.5 - 0.5 * x * y * y)
    return x * y


def _insert3(m1, m2, m3, t):
    m1, t = jnp.minimum(m1, t), jnp.maximum(m1, t)
    m2, t = jnp.minimum(m2, t), jnp.maximum(m2, t)
    m3 = jnp.minimum(m3, t)
    return m1, m2, m3


def _body(xs_h, ys_h, zs_h, sx_h, sy_h, sz_h, ox_h, oy_h, oz_h,
          xs_v, ys_v, zs_v, g1_v, g2_v, g3_v,
          sx_v, sy_v, sz_v, ox_v, oy_v, oz_v):
    wid = lax.axis_index("s") * 2 + lax.axis_index("c")
    base = wid * _QPW

    pltpu.sync_copy(xs_h, xs_v)
    pltpu.sync_copy(ys_h, ys_v)
    pltpu.sync_copy(zs_h, zs_v)
    pltpu.sync_copy(sx_h.at[pl.ds(base, _QPW)], sx_v)
    pltpu.sync_copy(sy_h.at[pl.ds(base, _QPW)], sy_v)
    pltpu.sync_copy(sz_h.at[pl.ds(base, _QPW)], sz_v)

    iota = lax.iota(jnp.int32, _L)
    lane0 = iota == 0
    inf_v = jnp.full((_L,), jnp.inf, jnp.float32)

    def per_pair(qp, _):
        qa = 2 * qp
        qb = qa + 1
        ia = jnp.full((_L,), base + qa, jnp.int32)
        ib = jnp.full((_L,), base + qb, jnp.int32)
        qxa = plsc.load_gather(xs_v, [ia])
        qya = plsc.load_gather(ys_v, [ia])
        qza = plsc.load_gather(zs_v, [ia])
        qxb = plsc.load_gather(xs_v, [ib])
        qyb = plsc.load_gather(ys_v, [ib])
        qzb = plsc.load_gather(zs_v, [ib])

        plsc.store_scatter(xs_v, [ia], inf_v, mask=lane0)
        plsc.store_scatter(xs_v, [ib], inf_v, mask=lane0)

        def chunk_body(c, carry):
            a1, a2, a3, b1, b2, b3 = carry
            coff = c * (_UNROLL * _L)
            for u in range(_UNROLL):
                off = coff + u * _L
                x = xs_v[pl.ds(off, _L)]
                y = ys_v[pl.ds(off, _L)]
                z = zs_v[pl.ds(off, _L)]
                dxa = qxa - x
                dya = qya - y
                dza = qza - z
                dxb = qxb - x
                dyb = qyb - y
                dzb = qzb - z
                ta = dxa * dxa + dya * dya + dza * dza
                tb = dxb * dxb + dyb * dyb + dzb * dzb
                a1, a2, a3 = _insert3(a1, a2, a3, ta)
                b1, b2, b3 = _insert3(b1, b2, b3, tb)
            return a1, a2, a3, b1, b2, b3

        a1, a2, a3, b1, b2, b3 = lax.fori_loop(
            0, _NCHUNK // _UNROLL, chunk_body,
            (inf_v, inf_v, inf_v, inf_v, inf_v, inf_v))

        plsc.store_scatter(xs_v, [ia], qxa, mask=lane0)
        plsc.store_scatter(xs_v, [ib], qxb, mask=lane0)

        dx = qxa - qxb
        dy = qya - qyb
        dz = qza - qzb
        dab = dx * dx + dy * dy + dz * dz
        for (m1, m2, m3, nm) in ((a1, a2, a3, "a"), (b1, b2, b3, "b")):
            i1, i2, i3 = _insert3(m1, m2, m3, dab)
            m1 = jnp.where(lane0, i1, m1)
            m2 = jnp.where(lane0, i2, m2)
            m3 = jnp.where(lane0, i3, m3)
            if nm == "a":
                a1, a2, a3 = m1, m2, m3
            else:
                b1, b2, b3 = m1, m2, m3

        for (m1, m2, m3, qfull) in ((a1, a2, a3, jnp.full((_L,), qa, jnp.int32)),
                                    (b1, b2, b3, jnp.full((_L,), qb, jnp.int32))):
            g = []
            for r in range(3):
                gr = jnp.min(m1)
                g.append(gr)
                if r < 2:
                    ff = plsc.all_reduce_ffs(m1 == gr)
                    sel = iota == ff
                    m1 = jnp.where(sel, m2, m1)
                    m2 = jnp.where(sel, m3, m2)
                    m3 = jnp.where(sel, inf_v, m3)
            plsc.store_scatter(g1_v, [qfull], jnp.full((_L,), g[0]), mask=lane0)
            plsc.store_scatter(g2_v, [qfull], jnp.full((_L,), g[1]), mask=lane0)
            plsc.store_scatter(g3_v, [qfull], jnp.full((_L,), g[2]), mask=lane0)
        return 0

    lax.fori_loop(0, _QPW // 2, per_pair, 0)

    for c in range(_QPW // _L):
        off = c * _L
        s = (_sqrt16(g1_v[pl.ds(off, _L)])
             + _sqrt16(g2_v[pl.ds(off, _L)])
             + _sqrt16(g3_v[pl.ds(off, _L)])) / 3.0
        s = jnp.maximum(s, 1e-5)
        ox_v[pl.ds(off, _L)] = s * sx_v[pl.ds(off, _L)]
        oy_v[pl.ds(off, _L)] = s * sy_v[pl.ds(off, _L)]
        oz_v[pl.ds(off, _L)] = s * sz_v[pl.ds(off, _L)]

    pltpu.sync_copy(ox_v, ox_h.at[pl.ds(base, _QPW)])
    pltpu.sync_copy(oy_v, oy_h.at[pl.ds(base, _QPW)])
    pltpu.sync_copy(oz_v, oz_h.at[pl.ds(base, _QPW)])


@jax.jit
def kernel(points, colors, scales):
    del colors
    xs = points[:, 0]
    ys = points[:, 1]
    zs = points[:, 2]
    sx = scales[:, 0]
    sy = scales[:, 1]
    sz = scales[:, 2]

    f32 = jnp.float32
    vec = jax.ShapeDtypeStruct((_N,), f32)
    run = pl.kernel(
        _body,
        out_type=(vec, vec, vec),
        mesh=plsc.VectorSubcoreMesh(core_axis_name="c", subcore_axis_name="s"),
        compiler_params=pltpu.CompilerParams(needs_layout_passes=False),
        scratch_types=(
            pltpu.VMEM((_N,), f32),
            pltpu.VMEM((_N,), f32),
            pltpu.VMEM((_N,), f32),
            pltpu.VMEM((_QPW,), f32),
            pltpu.VMEM((_QPW,), f32),
            pltpu.VMEM((_QPW,), f32),
            pltpu.VMEM((_QPW,), f32),
            pltpu.VMEM((_QPW,), f32),
            pltpu.VMEM((_QPW,), f32),
            pltpu.VMEM((_QPW,), f32),
            pltpu.VMEM((_QPW,), f32),
            pltpu.VMEM((_QPW,), f32),
        ),
    )
    ox, oy, oz = run(xs, ys, zs, sx, sy, sz)
    return jnp.stack([ox, oy, oz], axis=1)
